# baseline (speedup 1.0000x reference)
import functools

import jax
import jax.numpy as jnp
from jax import lax
from jax.experimental import pallas as pl
from jax.experimental.pallas import tpu as pltpu
from jax.experimental.pallas import tpu_sc as plsc

M = 500000
K = 250000
NC = 2
NS = 16
NW = NC * NS
HALF = M // NC

PZ = ((M + NS - 1) // NS + 127) // 128 * 128
PK = ((K + NS - 1) // NS + 127) // 128 * 128
PH = ((HALF + NS - 1) // NS + 7) // 8 * 8
PP = ((M + NW - 1) // NW + 15) // 16 * 16
REP = 245
assert NS * PZ >= M and (M - PZ) % 8 == 0
assert NS * PK >= K and (K - PK) % 8 == 0
assert NS * PH >= HALF and (HALF - PH) % 8 == 0
assert NW * PP >= M and (M - PP) % 8 == 0
assert PZ % (16 * REP) == 0 and PP % 8 == 0
NREP_Z = PZ // (16 * REP)
assert PP % (16 * REP) == 0 or PP < 16 * REP or True


@functools.partial(
    pl.kernel,
    out_type=(
        jax.ShapeDtypeStruct((M,), jnp.float32),
        jax.ShapeDtypeStruct((M,), jnp.float32),
        jax.ShapeDtypeStruct((M,), jnp.float32),
    ),
    mesh=plsc.VectorSubcoreMesh(
        core_axis_name="c", subcore_axis_name="s", num_cores=NC, num_subcores=NS
    ),
    scratch_types=[
        pltpu.VMEM_SHARED((M,), jnp.float32),
        pltpu.VMEM((PK,), jnp.int32),
        pltpu.VMEM((PK,), jnp.int32),
        pltpu.VMEM((PK,), jnp.float32),
        pltpu.VMEM((16 * REP,), jnp.float32),
        pltpu.VMEM((16 * REP,), jnp.float32),
        pltpu.VMEM((PH,), jnp.float32),
        pltpu.SemaphoreType.DMA,
        pltpu.SemaphoreType.DMA,
        pltpu.SemaphoreType.DMA,
        pltpu.SemaphoreType.DMA,
        pltpu.SemaphoreType.DMA,
        pltpu.SemaphoreType.DMA,
    ],
)
def _edge_mask_sc(perm_ref, probs_ref, soft_ref, hard_ref, mask_sh,
                  idx_v, idx2_v, one_v, zero_v, half_v, out_v,
                  sem_i, sem_z, sem_s, sem_p, sem_h, sem_h2):
    c = lax.axis_index("c")
    s = lax.axis_index("s")

    kbase = jnp.minimum(s * PK, K - PK)
    load = pltpu.async_copy(perm_ref.at[pl.ds(kbase, PK)], idx_v, sem_i)

    one = jnp.full((16,), 1.0, jnp.float32)
    zero = jnp.full((16,), 0.0, jnp.float32)
    half = jnp.full((16,), 0.5, jnp.float32)

    def fill_zero(i, _):
        zero_v[pl.ds(i * 16, 16)] = zero
        return 0

    lax.fori_loop(0, REP, fill_zero, 0, unroll=8)

    zbase = jnp.minimum(s * PZ, M - PZ)
    zcopies = [
        pltpu.async_copy(
            zero_v, mask_sh.at[pl.ds(zbase + j * 16 * REP, 16 * REP)], sem_z
        )
        for j in range(NREP_Z)
    ]

    def fill_half(i, _):
        half_v[pl.ds(i * 16, 16)] = half
        return 0

    lax.fori_loop(0, REP, fill_half, 0, unroll=8)

    def fill_one(i, _):
        one_v[pl.ds(i * 16, 16)] = one
        return 0

    lax.fori_loop(0, PK // 16, fill_one, 0, unroll=8)

    w = s * NC + c
    pbase = jnp.minimum(w * PP, M - PP)
    pcopies = []
    off = 0
    while off < PP:
        n = min(16 * REP, PP - off)
        pcopies.append(
            pltpu.async_copy(
                half_v.at[pl.ds(0, n)], probs_ref.at[pl.ds(pbase + off, n)],
                sem_p,
            )
        )
        off += n

    for zc in zcopies:
        zc.wait()
    plsc.subcore_barrier()
    plsc.subcore_barrier()

    load.wait()
    lo = c * HALF
    sentinel = jnp.full((16,), -1, jnp.int32)

    def filt(i, _):
        v = idx_v[pl.ds(i * 16, 16)]
        keep = (v >= lo) & (v < lo + HALF)
        idx2_v[pl.ds(i * 16, 16)] = jnp.where(keep, v, sentinel)
        return 0

    lax.fori_loop(0, PK // 16, filt, 0, unroll=8)
    scat = pltpu.async_copy(
        one_v, mask_sh.at[plsc.Indices(idx2_v, ignored_value=-1)], sem_s
    )
    scat.wait()
    plsc.subcore_barrier()
    plsc.subcore_barrier()

    hbase = c * HALF + jnp.minimum(s * PH, HALF - PH)
    H2 = PH // 2
    pltpu.sync_copy(mask_sh.at[pl.ds(hbase, H2)], out_v.at[pl.ds(0, H2)])
    out_h0 = pltpu.async_copy(
        out_v.at[pl.ds(0, H2)], hard_ref.at[pl.ds(hbase, H2)], sem_h
    )
    out_s0 = pltpu.async_copy(
        out_v.at[pl.ds(0, H2)], soft_ref.at[pl.ds(hbase, H2)], sem_h2
    )
    pltpu.sync_copy(
        mask_sh.at[pl.ds(hbase + H2, PH - H2)], out_v.at[pl.ds(H2, PH - H2)]
    )
    out_h1 = pltpu.async_copy(
        out_v.at[pl.ds(H2, PH - H2)], hard_ref.at[pl.ds(hbase + H2, PH - H2)],
        sem_h,
    )
    out_s1 = pltpu.async_copy(
        out_v.at[pl.ds(H2, PH - H2)], soft_ref.at[pl.ds(hbase + H2, PH - H2)],
        sem_h2,
    )
    for cp in pcopies:
        cp.wait()
    out_h0.wait()
    out_s0.wait()
    out_h1.wait()
    out_s1.wait()


def kernel(x, perm):
    del x
    probs, soft, hard = _edge_mask_sc(perm.astype(jnp.int32))
    return probs, soft, hard


# speedup vs baseline: 1.1808x; 1.1808x over previous
import functools

import jax
import jax.numpy as jnp
from jax import lax
from jax.experimental import pallas as pl
from jax.experimental.pallas import tpu as pltpu
from jax.experimental.pallas import tpu_sc as plsc

M = 500000
K = 250000
NC = 2
NS = 16
NW = NC * NS
HALF = M // NC

PZ = ((M + NS - 1) // NS + 127) // 128 * 128
PK = ((K + NS - 1) // NS + 127) // 128 * 128
PH = ((HALF + NS - 1) // NS + 7) // 8 * 8
PP = ((M + NW - 1) // NW + 15) // 16 * 16
REP = 245
assert NS * PZ >= M and (M - PZ) % 8 == 0
assert NS * PK >= K and (K - PK) % 8 == 0
assert NS * PH >= HALF and (HALF - PH) % 8 == 0
assert NW * PP >= M and (M - PP) % 8 == 0
assert PZ % (16 * REP) == 0 and PP % 8 == 0
NREP_Z = PZ // (16 * REP)
assert PP % (16 * REP) == 0 or PP < 16 * REP or True


@functools.partial(
    pl.kernel,
    out_type=(
        jax.ShapeDtypeStruct((M,), jnp.float32),
        jax.ShapeDtypeStruct((M,), jnp.float32),
        jax.ShapeDtypeStruct((M,), jnp.float32),
    ),
    mesh=plsc.VectorSubcoreMesh(
        core_axis_name="c", subcore_axis_name="s", num_cores=NC, num_subcores=NS
    ),
    scratch_types=[
        pltpu.VMEM_SHARED((M,), jnp.float32),
        pltpu.VMEM((PK,), jnp.int32),
        pltpu.VMEM((PK,), jnp.float32),
        pltpu.VMEM((16 * REP,), jnp.float32),
        pltpu.VMEM((16 * REP,), jnp.float32),
        pltpu.VMEM((PH,), jnp.float32),
        pltpu.SemaphoreType.DMA,
        pltpu.SemaphoreType.DMA,
        pltpu.SemaphoreType.DMA,
        pltpu.SemaphoreType.DMA,
        pltpu.SemaphoreType.DMA,
        pltpu.SemaphoreType.DMA,
    ],
)
def _edge_mask_sc(perm_ref, probs_ref, soft_ref, hard_ref, mask_sh,
                  idx_v, one_v, zero_v, half_v, out_v,
                  sem_i, sem_z, sem_s, sem_p, sem_h, sem_h2):
    c = lax.axis_index("c")
    s = lax.axis_index("s")

    kbase = jnp.minimum(s * PK, K - PK)
    load = pltpu.async_copy(perm_ref.at[pl.ds(kbase, PK)], idx_v, sem_i)

    one = jnp.full((16,), 1.0, jnp.float32)
    zero = jnp.full((16,), 0.0, jnp.float32)
    half = jnp.full((16,), 0.5, jnp.float32)

    def fill_zero(i, _):
        zero_v[pl.ds(i * 16, 16)] = zero
        return 0

    lax.fori_loop(0, REP, fill_zero, 0, unroll=8)

    zbase = jnp.minimum(s * PZ, M - PZ)
    zcopies = [
        pltpu.async_copy(
            zero_v, mask_sh.at[pl.ds(zbase + j * 16 * REP, 16 * REP)], sem_z
        )
        for j in range(NREP_Z)
    ]

    def fill_half(i, _):
        half_v[pl.ds(i * 16, 16)] = half
        return 0

    lax.fori_loop(0, REP, fill_half, 0, unroll=8)

    def fill_one(i, _):
        one_v[pl.ds(i * 16, 16)] = one
        return 0

    lax.fori_loop(0, PK // 16, fill_one, 0, unroll=8)

    w = s * NC + c
    pbase = jnp.minimum(w * PP, M - PP)
    pcopies = []
    off = 0
    while off < PP:
        n = min(16 * REP, PP - off)
        pcopies.append(
            pltpu.async_copy(
                half_v.at[pl.ds(0, n)], probs_ref.at[pl.ds(pbase + off, n)],
                sem_p,
            )
        )
        off += n

    for zc in zcopies:
        zc.wait()
    plsc.subcore_barrier()
    plsc.subcore_barrier()

    load.wait()
    scat = pltpu.async_copy(one_v, mask_sh.at[idx_v], sem_s)
    scat.wait()
    plsc.subcore_barrier()
    plsc.subcore_barrier()

    hbase = c * HALF + jnp.minimum(s * PH, HALF - PH)
    H2 = PH // 2
    pltpu.sync_copy(mask_sh.at[pl.ds(hbase, H2)], out_v.at[pl.ds(0, H2)])
    out_h0 = pltpu.async_copy(
        out_v.at[pl.ds(0, H2)], hard_ref.at[pl.ds(hbase, H2)], sem_h
    )
    out_s0 = pltpu.async_copy(
        out_v.at[pl.ds(0, H2)], soft_ref.at[pl.ds(hbase, H2)], sem_h2
    )
    pltpu.sync_copy(
        mask_sh.at[pl.ds(hbase + H2, PH - H2)], out_v.at[pl.ds(H2, PH - H2)]
    )
    out_h1 = pltpu.async_copy(
        out_v.at[pl.ds(H2, PH - H2)], hard_ref.at[pl.ds(hbase + H2, PH - H2)],
        sem_h,
    )
    out_s1 = pltpu.async_copy(
        out_v.at[pl.ds(H2, PH - H2)], soft_ref.at[pl.ds(hbase + H2, PH - H2)],
        sem_h2,
    )
    for cp in pcopies:
        cp.wait()
    out_h0.wait()
    out_s0.wait()
    out_h1.wait()
    out_s1.wait()


def kernel(x, perm):
    del x
    probs, soft, hard = _edge_mask_sc(perm.astype(jnp.int32))
    return probs, soft, hard
